# baseline (speedup 1.0000x reference)
import functools

import jax
import jax.numpy as jnp
from jax import lax
from jax.experimental import pallas as pl
from jax.experimental.pallas import tpu as pltpu
from jax.experimental.pallas import tpu_sc as plsc

_NC = 2
_NS = 16
_EB = 80
_RING = 5


def _segment_sum_sc(emb_cs, src3, dst3, zeros):
    n_pad, dh = zeros.shape
    rows_per_tile = src3.shape[1]
    stripe = n_pad // _NS

    mesh = plsc.VectorSubcoreMesh(core_axis_name="c", subcore_axis_name="s")

    @functools.partial(
        pl.kernel,
        out_type=jax.ShapeDtypeStruct((_NC, n_pad, dh), jnp.float32),
        mesh=mesh,
        scratch_types=[
            pltpu.VMEM((rows_per_tile, _EB), jnp.int32),
            pltpu.VMEM((rows_per_tile, _EB), jnp.int32),
            pltpu.VMEM((_RING, _EB, dh), jnp.float32),
            pltpu.VMEM_SHARED((n_pad, dh), jnp.float32),
        ] + [pltpu.SemaphoreType.DMA] * (2 * _RING),
        compiler_params=pltpu.CompilerParams(use_tc_tiling_on_sc=False),
    )
    def seg_sum(emb_hbm, src_hbm, dst_hbm, zero_hbm, out_hbm,
                src_v, dst_v, rows_v, agg_sh, *sems):
        c = lax.axis_index("c")
        s = lax.axis_index("s")
        pltpu.sync_copy(zero_hbm.at[pl.ds(s * stripe, stripe)],
                        agg_sh.at[pl.ds(s * stripe, stripe)])
        pltpu.sync_copy(src_hbm.at[s], src_v)
        pltpu.sync_copy(dst_hbm.at[s], dst_v)
        plsc.subcore_barrier()

        depth = _RING - 2
        half = emb_hbm.at[c]
        gsems = sems[:_RING]
        ssems = sems[_RING:]

        def fire(g, j):
            pltpu.async_copy(half.at[src_v.at[g]], rows_v.at[j], gsems[j])

        def drain_gather(j):
            pltpu.make_async_copy(half.at[src_v.at[0]], rows_v.at[j],
                                  gsems[j]).wait()

        def drain_scatter(j):
            pltpu.make_async_copy(rows_v.at[j], agg_sh.at[dst_v.at[0]],
                                  ssems[j]).wait()

        for j in range(depth):
            fire(j, j)

        def body(i, carry):
            for j in range(_RING):
                g = i * _RING + j
                gn = g + depth
                jn = (j + depth) % _RING

                @pl.when(gn < rows_per_tile)
                def _():
                    @pl.when(g >= _RING - depth)
                    def _():
                        drain_scatter(jn)

                    fire(gn, jn)

                drain_gather(j)
                pltpu.async_copy(rows_v.at[j], agg_sh.at[dst_v.at[g]],
                                 ssems[j], add=True)
            return carry

        lax.fori_loop(0, rows_per_tile // _RING, body, 0, unroll=False)
        for g in range(rows_per_tile - _RING, rows_per_tile):
            drain_scatter(g % _RING)
        plsc.subcore_barrier()
        pltpu.sync_copy(agg_sh.at[pl.ds(s * stripe, stripe)],
                        out_hbm.at[c, pl.ds(s * stripe, stripe)])

    return seg_sum(emb_cs, src3, dst3, zeros)


def _xo_body(x_ref, om_ref, b_ref, o_ref):
    o_ref[...] = jax.lax.dot_general(
        x_ref[...], om_ref[...], (((1,), (1,)), ((), ())),
        preferred_element_type=jnp.float32) + b_ref[...]


def _xo(x, om, b, block_rows=2000):
    n, d_in = x.shape
    d = om.shape[0]
    return pl.pallas_call(
        _xo_body,
        grid=(n // block_rows,),
        in_specs=[
            pl.BlockSpec((block_rows, d_in), lambda i: (i, 0)),
            pl.BlockSpec((d, d_in), lambda i: (0, 0)),
            pl.BlockSpec((1, d), lambda i: (0, 0)),
        ],
        out_specs=pl.BlockSpec((block_rows, d), lambda i: (i, 0)),
        out_shape=jax.ShapeDtypeStruct((n, d), jnp.float32),
    )(x, om, b)


def _head_body(p_ref, xo_ref, w_ref, hw_ref, hb_ref, o_ref):
    agg = jnp.concatenate((p_ref[0], p_ref[1]), axis=1)
    h = jax.lax.dot_general(agg, w_ref[...], (((1,), (1,)), ((), ())),
                            preferred_element_type=jnp.float32)
    h = jnp.maximum(h + xo_ref[...], 0.0)
    logits = jax.lax.dot_general(h, hw_ref[...], (((1,), (1,)), ((), ())),
                                 preferred_element_type=jnp.float32) + hb_ref[...]
    m = jnp.max(logits, axis=-1, keepdims=True)
    lse = jnp.log(jnp.sum(jnp.exp(logits - m), axis=-1, keepdims=True)) + m
    o_ref[...] = logits - lse


def _head(parts, xo, w, hw, hb, block_rows=2000):
    n, d = xo.shape
    dh = parts.shape[2]
    d_out = hw.shape[0]
    return pl.pallas_call(
        _head_body,
        grid=(n // block_rows,),
        in_specs=[
            pl.BlockSpec((_NC, block_rows, dh), lambda i: (0, i, 0)),
            pl.BlockSpec((block_rows, d), lambda i: (i, 0)),
            pl.BlockSpec((d, d), lambda i: (0, 0)),
            pl.BlockSpec((d_out, d), lambda i: (0, 0)),
            pl.BlockSpec((1, d_out), lambda i: (0, 0)),
        ],
        out_specs=pl.BlockSpec((block_rows, d_out), lambda i: (i, 0)),
        out_shape=jax.ShapeDtypeStruct((n, d_out), jnp.float32),
    )(parts, xo, w, hw, hb)


def kernel(node_index, x, edge_index, emb_table, W, Omega, b, head_W, head_b):
    n, d = emb_table.shape
    e = edge_index.shape[1]
    dh = d // _NC
    n_pad = -(-n // (8 * _NS)) * (8 * _NS)
    emb_cs = emb_table.reshape(n, _NC, dh).swapaxes(0, 1)
    src3 = edge_index[0].reshape(_NS, e // (_NS * _EB), _EB)
    dst3 = edge_index[1].reshape(_NS, e // (_NS * _EB), _EB)
    zeros = jnp.zeros((n_pad, dh), jnp.float32)
    parts = _segment_sum_sc(emb_cs, src3, dst3, zeros)
    xo = _xo(x, Omega, b.reshape(1, d))
    return _head(parts, xo, W, head_W, head_b.reshape(1, head_b.shape[0]))


# speedup vs baseline: 1.1189x; 1.1189x over previous
import functools

import jax
import jax.numpy as jnp
from jax import lax
from jax.experimental import pallas as pl
from jax.experimental.pallas import tpu as pltpu
from jax.experimental.pallas import tpu_sc as plsc

_NC = 2
_NS = 16
_EB = 80
_RING = 5


def _segment_sum_sc(emb_cs, sd4, n_pad):
    dh = emb_cs.shape[2]
    rows_per_tile = sd4.shape[2]
    stripe = n_pad // _NS

    mesh = plsc.VectorSubcoreMesh(core_axis_name="c", subcore_axis_name="s")

    @functools.partial(
        pl.kernel,
        out_type=jax.ShapeDtypeStruct((_NC, n_pad, dh), jnp.float32),
        mesh=mesh,
        scratch_types=[
            pltpu.VMEM((2, rows_per_tile, _EB), jnp.int32),
            pltpu.VMEM((_RING, _EB, dh), jnp.float32),
            pltpu.VMEM_SHARED((n_pad, dh), jnp.float32),
        ] + [pltpu.SemaphoreType.DMA] * _RING,
        compiler_params=pltpu.CompilerParams(use_tc_tiling_on_sc=False),
    )
    def seg_sum(emb_hbm, sd_hbm, out_hbm, sd_v, rows_v, agg_sh, *sems):
        c = lax.axis_index("c")
        s = lax.axis_index("s")
        def zbody(r, carry):
            for q in range(dh // 16):
                rows_v[0, r, pl.ds(q * 16, 16)] = jnp.zeros((16,), jnp.float32)
            return carry

        lax.fori_loop(0, _EB, zbody, 0, unroll=False)
        for r in range(stripe // _EB):
            pltpu.sync_copy(rows_v.at[0],
                            agg_sh.at[pl.ds(s * stripe + r * _EB, _EB)])
        pltpu.sync_copy(sd_hbm.at[:, s], sd_v)
        plsc.subcore_barrier()

        depth = _RING - 1
        half = emb_hbm.at[c]
        src_v = sd_v.at[0]
        dst_v = sd_v.at[1]

        def fire(g, j):
            pltpu.async_copy(half.at[src_v.at[g]], rows_v.at[j], sems[j])

        def drain(j):
            pltpu.make_async_copy(half.at[src_v.at[0]], rows_v.at[j],
                                  sems[j]).wait()

        for j in range(depth):
            fire(j, j)

        def body(i, carry):
            for j in range(_RING):
                g = i * _RING + j
                gn = g + depth
                jn = (j + depth) % _RING

                @pl.when(gn < rows_per_tile)
                def _():
                    fire(gn, jn)

                drain(j)
                pltpu.sync_copy(rows_v.at[j], agg_sh.at[dst_v.at[g]], add=True)
            return carry

        lax.fori_loop(0, rows_per_tile // _RING, body, 0, unroll=False)
        plsc.subcore_barrier()
        pltpu.sync_copy(agg_sh.at[pl.ds(s * stripe, stripe)],
                        out_hbm.at[c, pl.ds(s * stripe, stripe)])

    return seg_sum(emb_cs, sd4)


def _xo_body(x_ref, om_ref, b_ref, o_ref):
    o_ref[...] = jax.lax.dot_general(
        x_ref[...], om_ref[...], (((1,), (1,)), ((), ())),
        preferred_element_type=jnp.float32) + b_ref[...]


def _xo(x, om, b, block_rows=2000):
    n, d_in = x.shape
    d = om.shape[0]
    return pl.pallas_call(
        _xo_body,
        grid=(n // block_rows,),
        in_specs=[
            pl.BlockSpec((block_rows, d_in), lambda i: (i, 0)),
            pl.BlockSpec((d, d_in), lambda i: (0, 0)),
            pl.BlockSpec((1, d), lambda i: (0, 0)),
        ],
        out_specs=pl.BlockSpec((block_rows, d), lambda i: (i, 0)),
        out_shape=jax.ShapeDtypeStruct((n, d), jnp.float32),
    )(x, om, b)


def _head_body(p_ref, xo_ref, w_ref, hw_ref, hb_ref, o_ref):
    agg = jnp.concatenate((p_ref[0], p_ref[1]), axis=1)
    h = jax.lax.dot_general(agg, w_ref[...], (((1,), (1,)), ((), ())),
                            preferred_element_type=jnp.float32)
    h = jnp.maximum(h + xo_ref[...], 0.0)
    logits = jax.lax.dot_general(h, hw_ref[...], (((1,), (1,)), ((), ())),
                                 preferred_element_type=jnp.float32) + hb_ref[...]
    m = jnp.max(logits, axis=-1, keepdims=True)
    lse = jnp.log(jnp.sum(jnp.exp(logits - m), axis=-1, keepdims=True)) + m
    o_ref[...] = logits - lse


def _head(parts, xo, w, hw, hb, block_rows=2000):
    n, d = xo.shape
    dh = parts.shape[2]
    d_out = hw.shape[0]
    return pl.pallas_call(
        _head_body,
        grid=(n // block_rows,),
        in_specs=[
            pl.BlockSpec((_NC, block_rows, dh), lambda i: (0, i, 0)),
            pl.BlockSpec((block_rows, d), lambda i: (i, 0)),
            pl.BlockSpec((d, d), lambda i: (0, 0)),
            pl.BlockSpec((d_out, d), lambda i: (0, 0)),
            pl.BlockSpec((1, d_out), lambda i: (0, 0)),
        ],
        out_specs=pl.BlockSpec((block_rows, d_out), lambda i: (i, 0)),
        out_shape=jax.ShapeDtypeStruct((n, d_out), jnp.float32),
    )(parts, xo, w, hw, hb)


def kernel(node_index, x, edge_index, emb_table, W, Omega, b, head_W, head_b):
    n, d = emb_table.shape
    e = edge_index.shape[1]
    dh = d // _NC
    n_pad = -(-n // (8 * _NS)) * (8 * _NS)
    emb_cs = emb_table.reshape(n, _NC, dh).swapaxes(0, 1)
    sd4 = edge_index.reshape(2, _NS, e // (_NS * _EB), _EB)
    parts = _segment_sum_sc(emb_cs, sd4, n_pad)
    xo = _xo(x, Omega, b.reshape(1, d))
    return _head(parts, xo, W, head_W, head_b.reshape(1, head_b.shape[0]))
